# baseline (speedup 1.0000x reference)
import functools

import jax
import jax.numpy as jnp
from jax import lax
from jax.experimental import pallas as pl
from jax.experimental.pallas import tpu as pltpu
from jax.experimental.pallas import tpu_sc as plsc

D_MODEL_K = 4096
N_IN = 8
N_PROC = 32
TOPK = 3
BLK = 512

SC_CORES = 2
SC_SUBCORES = 16
SC_LANES = 16
NW = SC_CORES * SC_SUBCORES


def _router_body(x_ref, w_ref, w_out_ref, keys_out_ref):
    xb = x_ref[...]
    s = jax.lax.dot_general(
        xb, w_ref[...], (((1,), (0,)), ((), ())),
        preferred_element_type=jnp.float32)
    s_in = s[:, :N_IN]
    s_pr = s[:, N_IN:]

    m = jnp.max(s_in, axis=-1, keepdims=True)
    e = jnp.exp(s_in - m)
    w_out_ref[...] = e / jnp.sum(e, axis=-1, keepdims=True)

    iota = lax.broadcasted_iota(jnp.int32, s_pr.shape, 1)
    b = lax.bitcast_convert_type(s_pr, jnp.int32)
    keys = lax.bitcast_convert_type((b & ~31) | (31 - iota), jnp.float32)
    keys_out_ref[...] = keys.T


def _sc_topk_body(tpw, keys_hbm, out_hbm, keys_v, idx_v):
    wid = lax.axis_index("s") * SC_CORES + lax.axis_index("c")
    base = wid * tpw
    pltpu.sync_copy(keys_hbm.at[:, pl.ds(base, tpw)], keys_v)

    def group(g, carry):
        col = g * SC_LANES
        neg = jnp.full((SC_LANES,), -jnp.inf, jnp.float32)
        m1, m2, m3 = neg, neg, neg
        for e in range(N_PROC):
            t = keys_v[e, pl.ds(col, SC_LANES)]
            n1 = jnp.maximum(m1, t)
            t2 = jnp.minimum(m1, t)
            n2 = jnp.maximum(m2, t2)
            t3 = jnp.minimum(m2, t2)
            m3 = jnp.maximum(m3, t3)
            m1, m2 = n1, n2
        for k, mk in ((0, m1), (1, m2), (2, m3)):
            bk = lax.bitcast_convert_type(mk, jnp.int32)
            idx_v[k, pl.ds(col, SC_LANES)] = 31 - (bk & 31)
        return carry

    lax.fori_loop(0, tpw // SC_LANES, group, 0)
    pltpu.sync_copy(idx_v, out_hbm.at[:, pl.ds(base, tpw)])


def kernel(x, W_in, W_proc):
    B, S, D = x.shape
    T = B * S
    x2 = x.reshape(T, D)
    grid = (T // BLK,)
    weights, keys = pl.pallas_call(
        _router_body,
        grid=grid,
        in_specs=[
            pl.BlockSpec((BLK, D), lambda i: (i, 0)),
            pl.BlockSpec((D, N_IN + N_PROC), lambda i: (0, 0)),
        ],
        out_specs=[
            pl.BlockSpec((BLK, N_IN), lambda i: (i, 0)),
            pl.BlockSpec((N_PROC, BLK), lambda i: (0, i)),
        ],
        out_shape=[
            jax.ShapeDtypeStruct((T, N_IN), jnp.float32),
            jax.ShapeDtypeStruct((N_PROC, T), jnp.float32),
        ],
        compiler_params=pltpu.CompilerParams(
            dimension_semantics=("parallel",)),
    )(x2, jnp.concatenate([W_in, W_proc], axis=0).T)

    tpw = T // NW
    mesh = plsc.VectorSubcoreMesh(core_axis_name="c", subcore_axis_name="s")
    idx_t = pl.kernel(
        functools.partial(_sc_topk_body, tpw),
        mesh=mesh,
        out_type=jax.ShapeDtypeStruct((TOPK, T), jnp.int32),
        scratch_types=[
            pltpu.VMEM((N_PROC, tpw), jnp.float32),
            pltpu.VMEM((TOPK, tpw), jnp.int32),
        ],
    )(keys)

    return (idx_t.T.reshape(B, S, TOPK), weights.reshape(B, S, N_IN))


# speedup vs baseline: 1.0902x; 1.0902x over previous
import functools

import jax
import jax.numpy as jnp
from jax import lax
from jax.experimental import pallas as pl
from jax.experimental.pallas import tpu as pltpu
from jax.experimental.pallas import tpu_sc as plsc

D_MODEL_K = 4096
N_IN = 8
N_PROC = 32
TOPK = 3
BLK = 512

SC_CORES = 2
SC_SUBCORES = 16
SC_LANES = 16
NW = SC_CORES * SC_SUBCORES


def _score_block(xb, w, w_out_ref, keys_out_ref):
    s = jax.lax.dot_general(
        xb, w, (((1,), (0,)), ((), ())),
        preferred_element_type=jnp.float32)
    s_in = s[:, :N_IN]
    s_pr = s[:, N_IN:]

    m = jnp.max(s_in, axis=-1, keepdims=True)
    e = jnp.exp(s_in - m)
    w_out_ref[...] = e / jnp.sum(e, axis=-1, keepdims=True)

    iota = lax.broadcasted_iota(jnp.int32, s_pr.shape, 1)
    b = lax.bitcast_convert_type(s_pr, jnp.int32)
    keys = lax.bitcast_convert_type((b & ~31) | (31 - iota), jnp.float32)
    keys_out_ref[...] = keys.T


def _router_body(xa_ref, xb_ref, w_ref, wout_a, wout_b, keys_a, keys_b):
    w = w_ref[...]
    _score_block(xa_ref[...], w, wout_a, keys_a)
    _score_block(xb_ref[...], w, wout_b, keys_b)


def _sc_topk_body(tpw, nwh, keys_a_hbm, keys_b_hbm, out_hbm, keys_v, idx_v):
    wid = lax.axis_index("s") * SC_CORES + lax.axis_index("c")
    base = wid * tpw

    @pl.when(wid < nwh)
    def _():
        pltpu.sync_copy(keys_a_hbm.at[:, pl.ds(base, tpw)], keys_v)

    @pl.when(wid >= nwh)
    def _():
        pltpu.sync_copy(
            keys_b_hbm.at[:, pl.ds(base - nwh * tpw, tpw)], keys_v)

    def group(g, carry):
        col = g * SC_LANES
        neg = jnp.full((SC_LANES,), -jnp.inf, jnp.float32)
        m1, m2, m3 = neg, neg, neg
        for e in range(N_PROC):
            t = keys_v[e, pl.ds(col, SC_LANES)]
            n1 = jnp.maximum(m1, t)
            t2 = jnp.minimum(m1, t)
            n2 = jnp.maximum(m2, t2)
            t3 = jnp.minimum(m2, t2)
            m3 = jnp.maximum(m3, t3)
            m1, m2 = n1, n2
        for k, mk in ((0, m1), (1, m2), (2, m3)):
            bk = lax.bitcast_convert_type(mk, jnp.int32)
            idx_v[k, pl.ds(col, SC_LANES)] = 31 - (bk & 31)
        return carry

    lax.fori_loop(0, tpw // SC_LANES, group, 0)
    pltpu.sync_copy(idx_v, out_hbm.at[:, pl.ds(base, tpw)])


def kernel(x, W_in, W_proc):
    B, S, D = x.shape
    T = B * S
    x2 = x.reshape(T, D)
    half = T // (2 * BLK)
    Th = T // 2
    grid = (half,)
    wa, wb, keys_a, keys_b = pl.pallas_call(
        _router_body,
        grid=grid,
        in_specs=[
            pl.BlockSpec((BLK, D), lambda i: (i, 0)),
            pl.BlockSpec((BLK, D), lambda i: (i + half, 0)),
            pl.BlockSpec((D, N_IN + N_PROC), lambda i: (0, 0)),
        ],
        out_specs=[
            pl.BlockSpec((BLK, N_IN), lambda i: (i, 0)),
            pl.BlockSpec((BLK, N_IN), lambda i: (i, 0)),
            pl.BlockSpec((N_PROC, BLK), lambda i: (0, i)),
            pl.BlockSpec((N_PROC, BLK), lambda i: (0, i)),
        ],
        out_shape=[
            jax.ShapeDtypeStruct((Th, N_IN), jnp.float32),
            jax.ShapeDtypeStruct((Th, N_IN), jnp.float32),
            jax.ShapeDtypeStruct((N_PROC, Th), jnp.float32),
            jax.ShapeDtypeStruct((N_PROC, Th), jnp.float32),
        ],
        compiler_params=pltpu.CompilerParams(
            dimension_semantics=("parallel",)),
    )(x2, x2, jnp.concatenate([W_in, W_proc], axis=0).T)

    tpw = T // NW
    nwh = NW // 2
    mesh = plsc.VectorSubcoreMesh(core_axis_name="c", subcore_axis_name="s")
    idx_t = pl.kernel(
        functools.partial(_sc_topk_body, tpw, nwh),
        mesh=mesh,
        out_type=jax.ShapeDtypeStruct((TOPK, T), jnp.int32),
        scratch_types=[
            pltpu.VMEM((N_PROC, tpw), jnp.float32),
            pltpu.VMEM((TOPK, tpw), jnp.int32),
        ],
    )(keys_a, keys_b)

    weights = jnp.concatenate([wa, wb], axis=0)
    return (idx_t.T.reshape(B, S, TOPK), weights.reshape(B, S, N_IN))


# speedup vs baseline: 1.1753x; 1.0781x over previous
import functools

import jax
import jax.numpy as jnp
from jax import lax
from jax.experimental import pallas as pl
from jax.experimental.pallas import tpu as pltpu
from jax.experimental.pallas import tpu_sc as plsc

D_MODEL_K = 4096
N_IN = 8
N_PROC = 32
TOPK = 3
BLK = 512

SC_CORES = 2
SC_SUBCORES = 16
SC_LANES = 16
NW = SC_CORES * SC_SUBCORES


def _score_block(xb, w, w_out_ref, keys_out_ref):
    s = jax.lax.dot_general(
        w, xb, (((1,), (1,)), ((), ())),
        preferred_element_type=jnp.float32)
    s_in = s[:N_IN, :]
    s_pr = s[N_IN:, :]

    m = jnp.max(s_in, axis=0, keepdims=True)
    e = jnp.exp(s_in - m)
    w_out_ref[...] = e / jnp.sum(e, axis=0, keepdims=True)

    iota = lax.broadcasted_iota(jnp.int32, s_pr.shape, 0)
    b = lax.bitcast_convert_type(s_pr, jnp.int32)
    keys = lax.bitcast_convert_type((b & ~31) | (31 - iota), jnp.float32)
    keys_out_ref[...] = keys


def _router_body(xa_ref, xb_ref, wt_ref, wout_a, wout_b, keys_a, keys_b):
    wt = wt_ref[...]
    _score_block(xa_ref[...], wt, wout_a, keys_a)
    _score_block(xb_ref[...], wt, wout_b, keys_b)


def _sc_topk_body(tpw, nwh, keys_a_hbm, keys_b_hbm, out_hbm, keys_v, idx_v):
    wid = lax.axis_index("s") * SC_CORES + lax.axis_index("c")
    base = wid * tpw

    @pl.when(wid < nwh)
    def _():
        pltpu.sync_copy(keys_a_hbm.at[:, pl.ds(base, tpw)], keys_v)

    @pl.when(wid >= nwh)
    def _():
        pltpu.sync_copy(
            keys_b_hbm.at[:, pl.ds(base - nwh * tpw, tpw)], keys_v)

    def group(g, carry):
        col = g * SC_LANES
        neg = jnp.full((SC_LANES,), -jnp.inf, jnp.float32)
        m1, m2, m3 = neg, neg, neg
        for e in range(N_PROC):
            t = keys_v[e, pl.ds(col, SC_LANES)]
            n1 = jnp.maximum(m1, t)
            t2 = jnp.minimum(m1, t)
            n2 = jnp.maximum(m2, t2)
            t3 = jnp.minimum(m2, t2)
            m3 = jnp.maximum(m3, t3)
            m1, m2 = n1, n2
        for k, mk in ((0, m1), (1, m2), (2, m3)):
            bk = lax.bitcast_convert_type(mk, jnp.int32)
            idx_v[k, pl.ds(col, SC_LANES)] = 31 - (bk & 31)
        return carry

    lax.fori_loop(0, tpw // SC_LANES, group, 0)
    pltpu.sync_copy(idx_v, out_hbm.at[:, pl.ds(base, tpw)])


def kernel(x, W_in, W_proc):
    B, S, D = x.shape
    T = B * S
    x2 = x.reshape(T, D)
    half = T // (2 * BLK)
    Th = T // 2
    grid = (half,)
    wa, wb, keys_a, keys_b = pl.pallas_call(
        _router_body,
        grid=grid,
        in_specs=[
            pl.BlockSpec((BLK, D), lambda i: (i, 0)),
            pl.BlockSpec((BLK, D), lambda i: (i + half, 0)),
            pl.BlockSpec((N_IN + N_PROC, D), lambda i: (0, 0)),
        ],
        out_specs=[
            pl.BlockSpec((N_IN, BLK), lambda i: (0, i)),
            pl.BlockSpec((N_IN, BLK), lambda i: (0, i)),
            pl.BlockSpec((N_PROC, BLK), lambda i: (0, i)),
            pl.BlockSpec((N_PROC, BLK), lambda i: (0, i)),
        ],
        out_shape=[
            jax.ShapeDtypeStruct((N_IN, Th), jnp.float32),
            jax.ShapeDtypeStruct((N_IN, Th), jnp.float32),
            jax.ShapeDtypeStruct((N_PROC, Th), jnp.float32),
            jax.ShapeDtypeStruct((N_PROC, Th), jnp.float32),
        ],
        compiler_params=pltpu.CompilerParams(
            dimension_semantics=("parallel",)),
    )(x2, x2, jnp.concatenate([W_in, W_proc], axis=0))

    tpw = T // NW
    nwh = NW // 2
    mesh = plsc.VectorSubcoreMesh(core_axis_name="c", subcore_axis_name="s")
    idx_t = pl.kernel(
        functools.partial(_sc_topk_body, tpw, nwh),
        mesh=mesh,
        out_type=jax.ShapeDtypeStruct((TOPK, T), jnp.int32),
        scratch_types=[
            pltpu.VMEM((N_PROC, tpw), jnp.float32),
            pltpu.VMEM((TOPK, tpw), jnp.int32),
        ],
    )(keys_a, keys_b)

    weights = jnp.concatenate([wa, wb], axis=1).T
    return (idx_t.T.reshape(B, S, TOPK), weights.reshape(B, S, N_IN))


# speedup vs baseline: 1.1844x; 1.0077x over previous
import functools

import jax
import jax.numpy as jnp
from jax import lax
from jax.experimental import pallas as pl
from jax.experimental.pallas import tpu as pltpu
from jax.experimental.pallas import tpu_sc as plsc

D_MODEL_K = 4096
N_IN = 8
N_PROC = 32
TOPK = 3
BLK = 1024

SC_CORES = 2
SC_SUBCORES = 16
SC_LANES = 16
NW = SC_CORES * SC_SUBCORES


def _score_block(xb, w, w_out_ref, keys_out_ref):
    s = jax.lax.dot_general(
        w, xb, (((1,), (1,)), ((), ())),
        preferred_element_type=jnp.float32)
    s_in = s[:N_IN, :]
    s_pr = s[N_IN:, :]

    m = jnp.max(s_in, axis=0, keepdims=True)
    e = jnp.exp(s_in - m)
    w_out_ref[...] = e / jnp.sum(e, axis=0, keepdims=True)

    iota = lax.broadcasted_iota(jnp.int32, s_pr.shape, 0)
    b = lax.bitcast_convert_type(s_pr, jnp.int32)
    keys = lax.bitcast_convert_type((b & ~31) | (31 - iota), jnp.float32)
    keys_out_ref[...] = keys


def _router_body(xa_ref, wt_ref, wout_a, keys_a):
    _score_block(xa_ref[...], wt_ref[...], wout_a, keys_a)


def _sc_topk_body(tpw, keys_hbm, out_hbm, keys_v, idx_v):
    wid = lax.axis_index("s") * SC_CORES + lax.axis_index("c")
    base = wid * tpw
    pltpu.sync_copy(keys_hbm.at[:, pl.ds(base, tpw)], keys_v)

    def group(g, carry):
        col = g * SC_LANES
        neg = jnp.full((SC_LANES,), -jnp.inf, jnp.float32)
        m1, m2, m3 = neg, neg, neg
        for e in range(N_PROC):
            t = keys_v[e, pl.ds(col, SC_LANES)]
            n1 = jnp.maximum(m1, t)
            t2 = jnp.minimum(m1, t)
            n2 = jnp.maximum(m2, t2)
            t3 = jnp.minimum(m2, t2)
            m3 = jnp.maximum(m3, t3)
            m1, m2 = n1, n2
        for k, mk in ((0, m1), (1, m2), (2, m3)):
            bk = lax.bitcast_convert_type(mk, jnp.int32)
            idx_v[k, pl.ds(col, SC_LANES)] = 31 - (bk & 31)
        return carry

    lax.fori_loop(0, tpw // SC_LANES, group, 0)
    pltpu.sync_copy(idx_v, out_hbm.at[:, pl.ds(base, tpw)])


def kernel(x, W_in, W_proc):
    B, S, D = x.shape
    T = B * S
    x2 = x.reshape(T, D)
    grid = (T // BLK,)
    wt_all, keys = pl.pallas_call(
        _router_body,
        grid=grid,
        in_specs=[
            pl.BlockSpec((BLK, D), lambda i: (i, 0)),
            pl.BlockSpec((N_IN + N_PROC, D), lambda i: (0, 0)),
        ],
        out_specs=[
            pl.BlockSpec((N_IN, BLK), lambda i: (0, i)),
            pl.BlockSpec((N_PROC, BLK), lambda i: (0, i)),
        ],
        out_shape=[
            jax.ShapeDtypeStruct((N_IN, T), jnp.float32),
            jax.ShapeDtypeStruct((N_PROC, T), jnp.float32),
        ],
        compiler_params=pltpu.CompilerParams(
            dimension_semantics=("parallel",)),
    )(x2, jnp.concatenate([W_in, W_proc], axis=0))

    tpw = T // NW
    mesh = plsc.VectorSubcoreMesh(core_axis_name="c", subcore_axis_name="s")
    idx_t = pl.kernel(
        functools.partial(_sc_topk_body, tpw),
        mesh=mesh,
        out_type=jax.ShapeDtypeStruct((TOPK, T), jnp.int32),
        scratch_types=[
            pltpu.VMEM((N_PROC, tpw), jnp.float32),
            pltpu.VMEM((TOPK, tpw), jnp.int32),
        ],
    )(keys)

    weights = wt_all.T
    return (idx_t.T.reshape(B, S, TOPK), weights.reshape(B, S, N_IN))
